# Initial kernel scaffold; baseline (speedup 1.0000x reference)
import jax
import jax.numpy as jnp
from jax.experimental import pallas as pl


def kernel(x, edge_index, edge_weight, vertex_cnt, rule_cnt, W, b):
    raise NotImplementedError("write your pallas kernel here")


# speedup vs baseline: 4.5767x; 4.5767x over previous
import functools

import jax
import jax.numpy as jnp
from jax import lax
from jax.experimental import pallas as pl
from jax.experimental.pallas import tpu as pltpu
from jax.experimental.pallas import tpu_sc as plsc

NC = 2
NS = 16
L = 16
CHUNK = 128


def _sc_propagate(x, src, dst, ew, zrows):
    N, D = x.shape
    E = src.shape[0]
    NW = NC * NS
    EW = E // NW
    nchunk = EW // CHUNK
    assert E % NW == 0 and EW % CHUNK == 0 and D % L == 0
    assert nchunk % 2 == 0
    zn = zrows.shape[0]
    zlast = N - (NS - 1) * zn
    assert 0 < zlast <= zn and zn % 8 == 0 and zlast % 8 == 0

    mesh = plsc.VectorSubcoreMesh(
        core_axis_name="c", subcore_axis_name="s",
        num_cores=NC, num_subcores=NS)

    @functools.partial(
        pl.kernel,
        out_type=jax.ShapeDtypeStruct((NC, N, D), jnp.float32),
        mesh=mesh,
        compiler_params=pltpu.CompilerParams(needs_layout_passes=False),
        scratch_types=[
            pltpu.VMEM_SHARED((N, D), jnp.float32),
            pltpu.VMEM((CHUNK,), jnp.int32),
            pltpu.VMEM((CHUNK,), jnp.int32),
            pltpu.VMEM((CHUNK,), jnp.int32),
            pltpu.VMEM((CHUNK,), jnp.int32),
            pltpu.VMEM((CHUNK,), jnp.float32),
            pltpu.VMEM((CHUNK,), jnp.float32),
            pltpu.VMEM((CHUNK, D), jnp.float32),
            pltpu.VMEM((CHUNK, D), jnp.float32),
            pltpu.SemaphoreType.DMA,
            pltpu.SemaphoreType.DMA,
            pltpu.SemaphoreType.DMA,
            pltpu.SemaphoreType.DMA,
        ],
    )
    def kern(x_hbm, src_hbm, dst_hbm, ew_hbm, z_hbm, out_hbm,
             agg, sidx0, sidx1, didx0, didx1, wch0, wch1, rows0, rows1,
             semi0, semi1, semg0, semg1):
        c = lax.axis_index("c")
        s = lax.axis_index("s")
        wid = c * NS + s
        base = wid * EW

        sidx = (sidx0, sidx1)
        didx = (didx0, didx1)
        wch = (wch0, wch1)
        rows = (rows0, rows1)
        semi = (semi0, semi1)
        semg = (semg0, semg1)

        @pl.when(s < NS - 1)
        def _():
            pltpu.sync_copy(z_hbm, agg.at[pl.ds(s * zn, zn)])

        @pl.when(s == NS - 1)
        def _():
            pltpu.sync_copy(z_hbm.at[pl.ds(0, zlast)],
                            agg.at[pl.ds((NS - 1) * zn, zlast)])

        plsc.subcore_barrier()

        def idx_start(ci, b):
            off = base + ci * CHUNK
            pltpu.async_copy(src_hbm.at[pl.ds(off, CHUNK)], sidx[b], semi[b])
            pltpu.async_copy(dst_hbm.at[pl.ds(off, CHUNK)], didx[b], semi[b])
            pltpu.async_copy(ew_hbm.at[pl.ds(off, CHUNK)], wch[b], semi[b])

        def idx_wait(b):
            pltpu.make_async_copy(src_hbm.at[pl.ds(0, CHUNK)], sidx[b],
                                  semi[b]).wait()
            pltpu.make_async_copy(dst_hbm.at[pl.ds(0, CHUNK)], didx[b],
                                  semi[b]).wait()
            pltpu.make_async_copy(ew_hbm.at[pl.ds(0, CHUNK)], wch[b],
                                  semi[b]).wait()

        def gather_start(b):
            pltpu.make_async_copy(x_hbm.at[sidx[b]], rows[b], semg[b]).start()

        def gather_wait(b):
            pltpu.make_async_copy(x_hbm.at[sidx[b]], rows[b], semg[b]).wait()

        def process(b):
            @pl.loop(0, CHUNK)
            def _scale(r):
                wv = plsc.load_gather(wch[b], [jnp.full((L,), r, jnp.int32)])
                row = rows[b].at[r]
                for k in range(D // L):
                    sl = pl.ds(k * L, L)
                    row[sl] = row[sl] * wv

            pltpu.sync_copy(rows[b], agg.at[didx[b]], add=True)

        idx_start(0, 0)
        idx_start(1, 1)
        idx_wait(0)
        gather_start(0)

        @pl.loop(0, nchunk // 2)
        def _main(i):
            c0 = 2 * i
            last = i == nchunk // 2 - 1
            idx_wait(1)
            gather_start(1)
            gather_wait(0)
            process(0)

            @pl.when(jnp.logical_not(last))
            def _():
                idx_start(c0 + 2, 0)

            gather_wait(1)

            @pl.when(jnp.logical_not(last))
            def _():
                idx_wait(0)
                gather_start(0)

            process(1)

            @pl.when(jnp.logical_not(last))
            def _():
                idx_start(c0 + 3, 1)

        plsc.subcore_barrier()

        @pl.when(s < NS - 1)
        def _():
            pltpu.sync_copy(agg.at[pl.ds(s * zn, zn)],
                            out_hbm.at[c, pl.ds(s * zn, zn)])

        @pl.when(s == NS - 1)
        def _():
            pltpu.sync_copy(agg.at[pl.ds((NS - 1) * zn, zlast)],
                            out_hbm.at[c, pl.ds((NS - 1) * zn, zlast)])

    return kern(x, src, dst, ew, zrows)


def _tc_linear(partials, wt, b2):
    _, n, d = partials.shape
    bn = 1000
    assert n % bn == 0

    def body(p_ref, w_ref, b_ref, o_ref):
        acc = p_ref[0] + p_ref[1]
        o_ref[...] = jnp.dot(acc, w_ref[...],
                             preferred_element_type=jnp.float32) + b_ref[...]

    return pl.pallas_call(
        body,
        grid=(n // bn,),
        in_specs=[
            pl.BlockSpec((NC, bn, d), lambda i: (0, i, 0)),
            pl.BlockSpec((d, d), lambda i: (0, 0)),
            pl.BlockSpec((1, d), lambda i: (0, 0)),
        ],
        out_specs=pl.BlockSpec((bn, d), lambda i: (i, 0)),
        out_shape=jax.ShapeDtypeStruct((n, d), jnp.float32),
    )(partials, wt, b2)


def kernel(x, edge_index, edge_weight, vertex_cnt, rule_cnt, W, b):
    x = x.astype(jnp.float32)
    src = edge_index[0].astype(jnp.int32)
    dst = edge_index[1].astype(jnp.int32)
    ew = edge_weight.astype(jnp.float32)

    e = src.shape[0]
    unit = 2 * CHUNK * NC * NS
    ep = ((e + unit - 1) // unit) * unit
    pad = ep - e
    if pad:
        src = jnp.concatenate([src, jnp.zeros((pad,), jnp.int32)])
        dst = jnp.concatenate([dst, jnp.zeros((pad,), jnp.int32)])
        ew = jnp.concatenate([ew, jnp.zeros((pad,), jnp.float32)])

    n = x.shape[0]
    zn = ((n // NS + 7) // 8) * 8
    zrows = jnp.zeros((zn, x.shape[1]), jnp.float32)

    partials = _sc_propagate(x, src, dst, ew, zrows)
    return _tc_linear(partials, W.T.astype(jnp.float32),
                      b.reshape(1, -1).astype(jnp.float32))
